# baseline (speedup 1.0000x reference)
import functools

import jax
import jax.numpy as jnp
from jax import lax
from jax.experimental import pallas as pl
from jax.experimental.pallas import tpu as pltpu
from jax.experimental.pallas import tpu_sc as plsc

N = 10000
E = 320000
H = 128
G = 16
L = 4
BLK_N = 2000
BLK_E = 2000

NC = 2
NS = 16
NW = NC * NS
EPW = E // NW
CH = 80
NCH = EPW // CH
NPAD = 10240
RPT = NPAD // NS


def _silu(x):
    return x / (1.0 + jnp.exp(-x))


def _mm(a, b):
    return jax.lax.dot_general(
        a, b, (((1,), (0,)), ((), ())), preferred_element_type=jnp.float32
    )


def _embed_body(xpe, eW1, eb1, eW2, eb2, W1s, W1r, b1m, h_out, as_out, ar_out):
    t = _silu(_mm(xpe[...], eW1[...]) + eb1[...])
    h = _mm(t, eW2[...]) + eb2[...]
    h_out[...] = h
    as_out[...] = _mm(h, W1s[...]) + b1m[...]
    ar_out[...] = _mm(h, W1r[...])


def _upd_body(h, p0, p1, U1h, U1a, ub1, U2, ub2, W1s, W1r, b1m,
              h_out, as_out, ar_out):
    aggr = p0[...] + p1[...]
    t = _silu(_mm(h[...], U1h[...]) + _mm(aggr, U1a[...]) + ub1[...])
    hn = h[...] + _mm(t, U2[...]) + ub2[...]
    h_out[...] = hn
    as_out[...] = _mm(hn, W1s[...]) + b1m[...]
    ar_out[...] = _mm(hn, W1r[...])


def _upd_last_body(h, p0, p1, U1h, U1a, ub1, U2, ub2, h_out):
    aggr = p0[...] + p1[...]
    t = _silu(_mm(h[...], U1h[...]) + _mm(aggr, U1a[...]) + ub1[...])
    h_out[...] = h[...] + _mm(t, U2[...]) + ub2[...]


def _msg_body(pg, sq, wd, W2, b2, m_out):
    s = sq[...]
    dist = jnp.where(s > 0, jnp.sqrt(jnp.where(s > 0, s, 1.0)), 0.0)
    pre = pg[...] + dist * wd[...]
    t = _silu(pre)
    m_out[...] = _silu(_mm(t, W2[...]) + b2[...])


def _final_body(h, bt, pW1, pb1, pW2, pb2, rW1, rb1, rW2, rb2, out, acc):
    i = pl.program_id(0)

    @pl.when(i == 0)
    def _():
        acc[...] = jnp.zeros_like(acc)

    t = _silu(_mm(h[...], pW1[...]) + pb1[...])
    hp = _mm(t, pW2[...]) + pb2[...]
    oh = (bt[...] == jax.lax.broadcasted_iota(jnp.int32, (BLK_N, G), 1)
          ).astype(jnp.float32)
    acc[...] += jax.lax.dot_general(
        oh, hp, (((0,), (0,)), ((), ())), preferred_element_type=jnp.float32
    )

    @pl.when(i == pl.num_programs(0) - 1)
    def _():
        p = acc[...]
        t2 = _silu(_mm(p, rW1[...]) + rb1[...])
        out[...] = _mm(t2, rW2[...]) + rb2[...]


_SC_MESH = plsc.VectorSubcoreMesh(core_axis_name="c", subcore_axis_name="s")


def _gather_body(tabS, tabR, send3, rec3, out_p, idx_s, idx_r,
                 bSA, bRA, bOA, bSB, bRB, bOB, gsa, gsb, wsa, wsb):
    wid = lax.axis_index("c") * NS + lax.axis_index("s")
    base = wid * EPW
    pltpu.sync_copy(send3.at[wid], idx_s)
    pltpu.sync_copy(rec3.at[wid], idx_r)

    def fire_g(k, bS, bR, sem):
        pltpu.async_copy(tabS.at[idx_s.at[k]], bS, sem)
        pltpu.async_copy(tabR.at[idx_r.at[k]], bR, sem)

    def wait_g(k, bS, bR, sem):
        pltpu.make_async_copy(tabS.at[idx_s.at[k]], bS, sem).wait()
        pltpu.make_async_copy(tabR.at[idx_r.at[k]], bR, sem).wait()

    def fire_w(k, bO, sem):
        pltpu.async_copy(bO, out_p.at[pl.ds(base + k * CH, CH)], sem)

    def wait_w(k, bO, sem):
        pltpu.make_async_copy(
            bO, out_p.at[pl.ds(base + k * CH, CH)], sem).wait()

    def add(bS, bR, bO):
        def row(i, c):
            for j in range(8):
                sl = pl.ds(j * 16, 16)
                bO[i, sl] = bS[i, sl] + bR[i, sl]
            return c
        lax.fori_loop(0, CH, row, 0, unroll=2)

    fire_g(0, bSA, bRA, gsa)
    fire_g(1, bSB, bRB, gsb)

    def step(kk, carry):
        k0 = 2 * kk
        k1 = k0 + 1
        wait_g(k0, bSA, bRA, gsa)

        @pl.when(kk > 0)
        def _():
            wait_w(k0 - 2, bOA, wsa)

        add(bSA, bRA, bOA)
        fire_w(k0, bOA, wsa)
        fire_g(k0 + 2, bSA, bRA, gsa)

        wait_g(k1, bSB, bRB, gsb)

        @pl.when(kk > 0)
        def _():
            wait_w(k1 - 2, bOB, wsb)

        add(bSB, bRB, bOB)
        fire_w(k1, bOB, wsb)

        @pl.when(kk < (NCH - 1) // 2 - 1)
        def _():
            fire_g(k1 + 2, bSB, bRB, gsb)

        return carry

    lax.fori_loop(0, (NCH - 1) // 2, step, 0)
    k_last = NCH - 1
    wait_w(k_last - 2, bOA, wsa)
    wait_w(k_last - 1, bOB, wsb)
    wait_g(k_last, bSA, bRA, gsa)
    add(bSA, bRA, bOA)
    fire_w(k_last, bOA, wsa)
    wait_w(k_last, bOA, wsa)


_gather128 = pl.kernel(
    _gather_body,
    mesh=_SC_MESH,
    compiler_params=pltpu.CompilerParams(needs_layout_passes=False),
    out_type=jax.ShapeDtypeStruct((E, H), jnp.float32),
    scratch_types=[
        pltpu.VMEM((NCH, CH), jnp.int32),
        pltpu.VMEM((NCH, CH), jnp.int32),
        pltpu.VMEM((CH, H), jnp.float32),
        pltpu.VMEM((CH, H), jnp.float32),
        pltpu.VMEM((CH, H), jnp.float32),
        pltpu.VMEM((CH, H), jnp.float32),
        pltpu.VMEM((CH, H), jnp.float32),
        pltpu.VMEM((CH, H), jnp.float32),
        pltpu.SemaphoreType.DMA,
        pltpu.SemaphoreType.DMA,
        pltpu.SemaphoreType.DMA,
        pltpu.SemaphoreType.DMA,
    ],
)

_GPC = CH // 16


def _sq_body(px_h, py_h, pz_h, send3, rec3, sq_hbm, px, py, pz,
             idx_s, idx_r, buf):
    wid = lax.axis_index("c") * NS + lax.axis_index("s")
    base = wid * EPW
    pltpu.sync_copy(px_h, px)
    pltpu.sync_copy(py_h, py)
    pltpu.sync_copy(pz_h, pz)
    pltpu.sync_copy(send3.at[wid], idx_s)
    pltpu.sync_copy(rec3.at[wid], idx_r)

    def chunk(k, carry):
        for j in range(_GPC):
            ivs = idx_s[k, pl.ds(j * 16, 16)]
            ivr = idx_r[k, pl.ds(j * 16, 16)]
            dx = plsc.load_gather(px, [ivs]) - plsc.load_gather(px, [ivr])
            dy = plsc.load_gather(py, [ivs]) - plsc.load_gather(py, [ivr])
            dz = plsc.load_gather(pz, [ivs]) - plsc.load_gather(pz, [ivr])
            buf[pl.ds(j * 16, 16)] = dx * dx + dy * dy + dz * dz
        pltpu.sync_copy(buf, sq_hbm.at[pl.ds(base + k * CH, CH)])
        return carry

    lax.fori_loop(0, NCH, chunk, 0)


_sq_call = pl.kernel(
    _sq_body,
    mesh=_SC_MESH,
    compiler_params=pltpu.CompilerParams(needs_layout_passes=False),
    out_type=jax.ShapeDtypeStruct((E,), jnp.float32),
    scratch_types=[
        pltpu.VMEM((N,), jnp.float32),
        pltpu.VMEM((N,), jnp.float32),
        pltpu.VMEM((N,), jnp.float32),
        pltpu.VMEM((NCH, CH), jnp.int32),
        pltpu.VMEM((NCH, CH), jnp.int32),
        pltpu.VMEM((CH,), jnp.float32),
    ],
)


def _scatter_body(m_hbm, rec3, zeros_hbm, out_hbm, idx_r, bufA, bufB,
                  aggr_sh, sA, sB):
    cid = lax.axis_index("c")
    sid = lax.axis_index("s")
    wid = cid * NS + sid
    base = wid * EPW
    pltpu.sync_copy(zeros_hbm.at[pl.ds(sid * RPT, RPT)],
                    aggr_sh.at[pl.ds(sid * RPT, RPT)])
    pltpu.sync_copy(rec3.at[wid], idx_r)
    plsc.subcore_barrier()

    def fire(k, buf, sem):
        pltpu.async_copy(m_hbm.at[pl.ds(base + k * CH, CH)], buf, sem)

    def wait(k, buf, sem):
        pltpu.make_async_copy(
            m_hbm.at[pl.ds(base + k * CH, CH)], buf, sem).wait()

    fire(0, bufA, sA)

    def step(kk, carry):
        k0 = 2 * kk
        k1 = k0 + 1
        wait(k0, bufA, sA)
        fire(k1, bufB, sB)
        pltpu.sync_copy(bufA, aggr_sh.at[idx_r.at[k0]], add=True)
        wait(k1, bufB, sB)
        fire(k0 + 2, bufA, sA)
        pltpu.sync_copy(bufB, aggr_sh.at[idx_r.at[k1]], add=True)
        return carry

    lax.fori_loop(0, (NCH - 1) // 2, step, 0)
    k_last = NCH - 1
    wait(k_last, bufA, sA)
    pltpu.sync_copy(bufA, aggr_sh.at[idx_r.at[k_last]], add=True)
    plsc.subcore_barrier()
    pltpu.sync_copy(aggr_sh.at[pl.ds(sid * RPT, RPT)],
                    out_hbm.at[cid, pl.ds(sid * RPT, RPT)])


_scatter_call = pl.kernel(
    _scatter_body,
    mesh=_SC_MESH,
    out_type=jax.ShapeDtypeStruct((NC, NPAD, H), jnp.float32),
    scratch_types=[
        pltpu.VMEM((NCH, CH), jnp.int32),
        pltpu.VMEM((CH, H), jnp.float32),
        pltpu.VMEM((CH, H), jnp.float32),
        pltpu.VMEM_SHARED((NPAD, H), jnp.float32),
        pltpu.SemaphoreType.DMA,
        pltpu.SemaphoreType.DMA,
    ],
)


def _full(shape):
    return pl.BlockSpec(shape, lambda i: (0,) * len(shape))


def _nrows(width):
    return pl.BlockSpec((BLK_N, width), lambda i: (i, 0))


def _erows(width):
    return pl.BlockSpec((BLK_E, width), lambda i: (i, 0))


_W = _full((H, H))
_B = _full((1, H))

_embed_call = pl.pallas_call(
    _embed_body,
    grid=(N // BLK_N,),
    in_specs=[_nrows(H), _W, _B, _W, _B, _W, _W, _B],
    out_specs=[_nrows(H), _nrows(H), _nrows(H)],
    out_shape=[jax.ShapeDtypeStruct((N, H), jnp.float32)] * 3,
)

_upd_call = pl.pallas_call(
    _upd_body,
    grid=(N // BLK_N,),
    in_specs=[_nrows(H)] * 3 + [_W, _W, _B, _W, _B, _W, _W, _B],
    out_specs=[_nrows(H), _nrows(H), _nrows(H)],
    out_shape=[jax.ShapeDtypeStruct((N, H), jnp.float32)] * 3,
)

_upd_last_call = pl.pallas_call(
    _upd_last_body,
    grid=(N // BLK_N,),
    in_specs=[_nrows(H)] * 3 + [_W, _W, _B, _W, _B],
    out_specs=_nrows(H),
    out_shape=jax.ShapeDtypeStruct((N, H), jnp.float32),
)

_msg_call = pl.pallas_call(
    _msg_body,
    grid=(E // BLK_E,),
    in_specs=[_erows(H), _erows(1), _B, _W, _B],
    out_specs=_erows(H),
    out_shape=jax.ShapeDtypeStruct((E, H), jnp.float32),
)

_final_call = pl.pallas_call(
    _final_body,
    grid=(N // BLK_N,),
    in_specs=[_nrows(H), _nrows(1), _W, _B, _W, _B, _W, _B, _full((H, 1)),
              _full((1, 1))],
    out_specs=_full((G, 1)),
    out_shape=jax.ShapeDtypeStruct((G, 1), jnp.float32),
    scratch_shapes=[pltpu.VMEM((G, H), jnp.float32)],
)


def kernel(x, pos, pe, edge_index, batch,
           embed_W1, embed_b1, embed_W2, embed_b2,
           msg_W1, msg_b1, msg_W2, msg_b2,
           upd_W1, upd_b1, upd_W2, upd_b2,
           pre_W1, pre_b1, pre_W2, pre_b2,
           ro_W1, ro_b1, ro_W2, ro_b2):
    xpe = jnp.concatenate([x, pe], axis=1)
    send, rec = edge_index[0], edge_index[1]
    send3 = send.reshape(NW, NCH, CH)
    rec3 = rec.reshape(NW, NCH, CH)

    sq = _sq_call(pos[:, 0], pos[:, 1], pos[:, 2], send3, rec3)[:, None]

    W1s = [msg_W1[l, :H] for l in range(L)]
    W1r = [msg_W1[l, H:2 * H] for l in range(L)]
    wd = [msg_W1[l, 2 * H][None] for l in range(L)]
    b1m = [msg_b1[l][None] for l in range(L)]
    U1h = [upd_W1[l, :H] for l in range(L)]
    U1a = [upd_W1[l, H:] for l in range(L)]
    ub1 = [upd_b1[l][None] for l in range(L)]
    ub2 = [upd_b2[l][None] for l in range(L)]

    h, As, Ar = _embed_call(
        xpe, embed_W1, embed_b1[None], embed_W2, embed_b2[None],
        W1s[0], W1r[0], b1m[0])

    zeros_n = jnp.zeros((NPAD, H), jnp.float32)
    for l in range(L):
        P = _gather128(As, Ar, send3, rec3)
        m = _msg_call(P, sq, wd[l], msg_W2[l], msg_b2[l][None])
        parts = _scatter_call(m, rec3, zeros_n)
        if l < L - 1:
            h, As, Ar = _upd_call(
                h, parts[0], parts[1], U1h[l], U1a[l], ub1[l], upd_W2[l],
                ub2[l], W1s[l + 1], W1r[l + 1], b1m[l + 1])
        else:
            h = _upd_last_call(
                h, parts[0], parts[1], U1h[l], U1a[l], ub1[l], upd_W2[l],
                ub2[l])

    out = _final_call(
        h, batch[:, None], pre_W1, pre_b1[None], pre_W2, pre_b2[None],
        ro_W1, ro_b1[None], ro_W2, ro_b2[None])
    return jnp.squeeze(out)


# speedup vs baseline: 1.1499x; 1.1499x over previous
import functools

import jax
import jax.numpy as jnp
from jax import lax
from jax.experimental import pallas as pl
from jax.experimental.pallas import tpu as pltpu
from jax.experimental.pallas import tpu_sc as plsc

N = 10000
E = 320000
H = 128
G = 16
L = 4
BLK_N = 2000
BLK_E = 2000

NC = 2
NS = 16
NW = NC * NS
EPW = E // NW
CH = 80
NCH = EPW // CH
NPAD = 10240
RPT = NPAD // NS


def _silu(x):
    return x / (1.0 + jnp.exp(-x))


def _mm(a, b):
    return jax.lax.dot_general(
        a, b, (((1,), (0,)), ((), ())), preferred_element_type=jnp.float32
    )


def _embed_body(xpe, eW1, eb1, eW2, eb2, W1s, W1r, b1m, h_out, as_out, ar_out):
    t = _silu(_mm(xpe[...], eW1[...]) + eb1[...])
    h = _mm(t, eW2[...]) + eb2[...]
    h_out[...] = h
    as_out[...] = _mm(h, W1s[...]) + b1m[...]
    ar_out[...] = _mm(h, W1r[...])


def _upd_body(h, p0, p1, p2, p3, U1h, U1a, ub1, U2, ub2, W1s, W1r, b1m,
              h_out, as_out, ar_out):
    aggr = (p0[...] + p1[...]) + (p2[...] + p3[...])
    t = _silu(_mm(h[...], U1h[...]) + _mm(aggr, U1a[...]) + ub1[...])
    hn = h[...] + _mm(t, U2[...]) + ub2[...]
    h_out[...] = hn
    as_out[...] = _mm(hn, W1s[...]) + b1m[...]
    ar_out[...] = _mm(hn, W1r[...])


def _upd_last_body(h, p0, p1, p2, p3, U1h, U1a, ub1, U2, ub2, h_out):
    aggr = (p0[...] + p1[...]) + (p2[...] + p3[...])
    t = _silu(_mm(h[...], U1h[...]) + _mm(aggr, U1a[...]) + ub1[...])
    h_out[...] = h[...] + _mm(t, U2[...]) + ub2[...]


def _msg_body(asg, brg, sq, wd, W2, b2, m_out):
    s = sq[...]
    dist = jnp.where(s > 0, jnp.sqrt(jnp.where(s > 0, s, 1.0)), 0.0)
    pre = asg[...] + brg[...] + dist * wd[...]
    t = _silu(pre)
    m_out[...] = _silu(_mm(t, W2[...]) + b2[...])


def _final_body(h, bt, pW1, pb1, pW2, pb2, rW1, rb1, rW2, rb2, out, acc):
    i = pl.program_id(0)

    @pl.when(i == 0)
    def _():
        acc[...] = jnp.zeros_like(acc)

    t = _silu(_mm(h[...], pW1[...]) + pb1[...])
    hp = _mm(t, pW2[...]) + pb2[...]
    oh = (bt[...] == jax.lax.broadcasted_iota(jnp.int32, (BLK_N, G), 1)
          ).astype(jnp.float32)
    acc[...] += jax.lax.dot_general(
        oh, hp, (((0,), (0,)), ((), ())), preferred_element_type=jnp.float32
    )

    @pl.when(i == pl.num_programs(0) - 1)
    def _():
        p = acc[...]
        t2 = _silu(_mm(p, rW1[...]) + rb1[...])
        out[...] = _mm(t2, rW2[...]) + rb2[...]


_SC_MESH = plsc.VectorSubcoreMesh(core_axis_name="c", subcore_axis_name="s")


def _make_gather(ne, ch):
  epw = ne // NW
  nch = epw // ch

  def _gather_body(tabS, tabR, send3, rec3, out0, out1, idx_s, idx_r,
                   bSA, bRA, bSB, bRB, gsa, gsb, wsa, wsb):
    wid = lax.axis_index("c") * NS + lax.axis_index("s")
    base = wid * epw
    pltpu.sync_copy(send3.at[wid], idx_s)
    pltpu.sync_copy(rec3.at[wid], idx_r)

    def fire_g(k, bS, bR, sem):
        pltpu.async_copy(tabS.at[idx_s.at[k]], bS, sem)
        pltpu.async_copy(tabR.at[idx_r.at[k]], bR, sem)

    def wait_g(k, bS, bR, sem):
        pltpu.make_async_copy(tabS.at[idx_s.at[k]], bS, sem).wait()
        pltpu.make_async_copy(tabR.at[idx_r.at[k]], bR, sem).wait()

    def fire_w(k, bS, bR, sem):
        pltpu.async_copy(bS, out0.at[pl.ds(base + k * ch, ch)], sem)
        pltpu.async_copy(bR, out1.at[pl.ds(base + k * ch, ch)], sem)

    def wait_w(k, bS, bR, sem):
        pltpu.make_async_copy(
            bS, out0.at[pl.ds(base + k * ch, ch)], sem).wait()
        pltpu.make_async_copy(
            bR, out1.at[pl.ds(base + k * ch, ch)], sem).wait()

    fire_g(0, bSA, bRA, gsa)
    fire_g(1, bSB, bRB, gsb)

    def step(kk, carry):
        k0 = 2 * kk
        k1 = k0 + 1
        wait_g(k0, bSA, bRA, gsa)
        fire_w(k0, bSA, bRA, wsa)
        wait_g(k1, bSB, bRB, gsb)
        fire_w(k1, bSB, bRB, wsb)
        wait_w(k0, bSA, bRA, wsa)
        fire_g(k0 + 2, bSA, bRA, gsa)
        wait_w(k1, bSB, bRB, wsb)

        @pl.when(kk < (nch - 1) // 2 - 1)
        def _():
            fire_g(k1 + 2, bSB, bRB, gsb)

        return carry

    lax.fori_loop(0, (nch - 1) // 2, step, 0)
    k_last = nch - 1
    wait_g(k_last, bSA, bRA, gsa)
    fire_w(k_last, bSA, bRA, wsa)
    wait_w(k_last, bSA, bRA, wsa)

  return pl.kernel(
      _gather_body,
      mesh=_SC_MESH,
      out_type=[jax.ShapeDtypeStruct((ne, H), jnp.float32)] * 2,
      scratch_types=[
          pltpu.VMEM((nch, ch), jnp.int32),
          pltpu.VMEM((nch, ch), jnp.int32),
          pltpu.VMEM((ch, H), jnp.float32),
          pltpu.VMEM((ch, H), jnp.float32),
          pltpu.VMEM((ch, H), jnp.float32),
          pltpu.VMEM((ch, H), jnp.float32),
          pltpu.SemaphoreType.DMA,
          pltpu.SemaphoreType.DMA,
          pltpu.SemaphoreType.DMA,
          pltpu.SemaphoreType.DMA,
      ],
  )


EH = E // 2
CHH = 40
_gather_h = _make_gather(EH, CHH)

_GPC = CH // 16


def _sq_body(px_h, py_h, pz_h, send3, rec3, sq_hbm, px, py, pz,
             idx_s, idx_r, buf):
    wid = lax.axis_index("c") * NS + lax.axis_index("s")
    base = wid * EPW
    pltpu.sync_copy(px_h, px)
    pltpu.sync_copy(py_h, py)
    pltpu.sync_copy(pz_h, pz)
    pltpu.sync_copy(send3.at[wid], idx_s)
    pltpu.sync_copy(rec3.at[wid], idx_r)

    def chunk(k, carry):
        for j in range(_GPC):
            ivs = idx_s[k, pl.ds(j * 16, 16)]
            ivr = idx_r[k, pl.ds(j * 16, 16)]
            dx = plsc.load_gather(px, [ivs]) - plsc.load_gather(px, [ivr])
            dy = plsc.load_gather(py, [ivs]) - plsc.load_gather(py, [ivr])
            dz = plsc.load_gather(pz, [ivs]) - plsc.load_gather(pz, [ivr])
            buf[pl.ds(j * 16, 16)] = dx * dx + dy * dy + dz * dz
        pltpu.sync_copy(buf, sq_hbm.at[pl.ds(base + k * CH, CH)])
        return carry

    lax.fori_loop(0, NCH, chunk, 0)


_sq_call = pl.kernel(
    _sq_body,
    mesh=_SC_MESH,
    compiler_params=pltpu.CompilerParams(needs_layout_passes=False),
    out_type=jax.ShapeDtypeStruct((E,), jnp.float32),
    scratch_types=[
        pltpu.VMEM((N,), jnp.float32),
        pltpu.VMEM((N,), jnp.float32),
        pltpu.VMEM((N,), jnp.float32),
        pltpu.VMEM((NCH, CH), jnp.int32),
        pltpu.VMEM((NCH, CH), jnp.int32),
        pltpu.VMEM((CH,), jnp.float32),
    ],
)


def _make_scatter(ne, ch):
  epw = ne // NW
  nch = epw // ch

  def _scatter_body(m_hbm, rec3, zeros_hbm, out_hbm, idx_r, bufA, bufB,
                    aggr_sh, sA, sB):
    cid = lax.axis_index("c")
    sid = lax.axis_index("s")
    wid = cid * NS + sid
    base = wid * epw
    pltpu.sync_copy(zeros_hbm.at[pl.ds(sid * RPT, RPT)],
                    aggr_sh.at[pl.ds(sid * RPT, RPT)])
    pltpu.sync_copy(rec3.at[wid], idx_r)
    plsc.subcore_barrier()

    def fire(k, buf, sem):
        pltpu.async_copy(m_hbm.at[pl.ds(base + k * ch, ch)], buf, sem)

    def wait(k, buf, sem):
        pltpu.make_async_copy(
            m_hbm.at[pl.ds(base + k * ch, ch)], buf, sem).wait()

    fire(0, bufA, sA)

    def step(kk, carry):
        k0 = 2 * kk
        k1 = k0 + 1
        wait(k0, bufA, sA)
        fire(k1, bufB, sB)
        pltpu.sync_copy(bufA, aggr_sh.at[idx_r.at[k0]], add=True)
        wait(k1, bufB, sB)
        fire(k0 + 2, bufA, sA)
        pltpu.sync_copy(bufB, aggr_sh.at[idx_r.at[k1]], add=True)
        return carry

    lax.fori_loop(0, (nch - 1) // 2, step, 0)
    k_last = nch - 1
    wait(k_last, bufA, sA)
    pltpu.sync_copy(bufA, aggr_sh.at[idx_r.at[k_last]], add=True)
    plsc.subcore_barrier()
    pltpu.sync_copy(aggr_sh.at[pl.ds(sid * RPT, RPT)],
                    out_hbm.at[cid, pl.ds(sid * RPT, RPT)])

  return pl.kernel(
      _scatter_body,
      mesh=_SC_MESH,
      out_type=jax.ShapeDtypeStruct((NC, NPAD, H), jnp.float32),
      scratch_types=[
          pltpu.VMEM((nch, ch), jnp.int32),
          pltpu.VMEM((ch, H), jnp.float32),
          pltpu.VMEM((ch, H), jnp.float32),
          pltpu.VMEM_SHARED((NPAD, H), jnp.float32),
          pltpu.SemaphoreType.DMA,
          pltpu.SemaphoreType.DMA,
      ],
  )


_scatter_h = _make_scatter(EH, CHH)


def _full(shape):
    return pl.BlockSpec(shape, lambda i: (0,) * len(shape))


def _nrows(width):
    return pl.BlockSpec((BLK_N, width), lambda i: (i, 0))


def _erows(width):
    return pl.BlockSpec((BLK_E, width), lambda i: (i, 0))


_W = _full((H, H))
_B = _full((1, H))

_embed_call = pl.pallas_call(
    _embed_body,
    grid=(N // BLK_N,),
    in_specs=[_nrows(H), _W, _B, _W, _B, _W, _W, _B],
    out_specs=[_nrows(H), _nrows(H), _nrows(H)],
    out_shape=[jax.ShapeDtypeStruct((N, H), jnp.float32)] * 3,
)

_upd_call = pl.pallas_call(
    _upd_body,
    grid=(N // BLK_N,),
    in_specs=[_nrows(H)] * 5 + [_W, _W, _B, _W, _B, _W, _W, _B],
    out_specs=[_nrows(H), _nrows(H), _nrows(H)],
    out_shape=[jax.ShapeDtypeStruct((N, H), jnp.float32)] * 3,
)

_upd_last_call = pl.pallas_call(
    _upd_last_body,
    grid=(N // BLK_N,),
    in_specs=[_nrows(H)] * 5 + [_W, _W, _B, _W, _B],
    out_specs=_nrows(H),
    out_shape=jax.ShapeDtypeStruct((N, H), jnp.float32),
)

_msg_call = pl.pallas_call(
    _msg_body,
    grid=(EH // BLK_E,),
    in_specs=[_erows(H), _erows(H), _erows(1), _B, _W, _B],
    out_specs=_erows(H),
    out_shape=jax.ShapeDtypeStruct((EH, H), jnp.float32),
)

_final_call = pl.pallas_call(
    _final_body,
    grid=(N // BLK_N,),
    in_specs=[_nrows(H), _nrows(1), _W, _B, _W, _B, _W, _B, _full((H, 1)),
              _full((1, 1))],
    out_specs=_full((G, 1)),
    out_shape=jax.ShapeDtypeStruct((G, 1), jnp.float32),
    scratch_shapes=[pltpu.VMEM((G, H), jnp.float32)],
)


def kernel(x, pos, pe, edge_index, batch,
           embed_W1, embed_b1, embed_W2, embed_b2,
           msg_W1, msg_b1, msg_W2, msg_b2,
           upd_W1, upd_b1, upd_W2, upd_b2,
           pre_W1, pre_b1, pre_W2, pre_b2,
           ro_W1, ro_b1, ro_W2, ro_b2):
    xpe = jnp.concatenate([x, pe], axis=1)
    send, rec = edge_index[0], edge_index[1]
    send3 = send.reshape(NW, NCH, CH)
    rec3 = rec.reshape(NW, NCH, CH)
    nchh = (EH // NW) // CHH
    send3h = [send[:EH].reshape(NW, nchh, CHH), send[EH:].reshape(NW, nchh, CHH)]
    rec3h = [rec[:EH].reshape(NW, nchh, CHH), rec[EH:].reshape(NW, nchh, CHH)]

    sq = _sq_call(pos[:, 0], pos[:, 1], pos[:, 2], send3, rec3)[:, None]

    W1s = [msg_W1[l, :H] for l in range(L)]
    W1r = [msg_W1[l, H:2 * H] for l in range(L)]
    wd = [msg_W1[l, 2 * H][None] for l in range(L)]
    b1m = [msg_b1[l][None] for l in range(L)]
    U1h = [upd_W1[l, :H] for l in range(L)]
    U1a = [upd_W1[l, H:] for l in range(L)]
    ub1 = [upd_b1[l][None] for l in range(L)]
    ub2 = [upd_b2[l][None] for l in range(L)]

    h, As, Ar = _embed_call(
        xpe, embed_W1, embed_b1[None], embed_W2, embed_b2[None],
        W1s[0], W1r[0], b1m[0])

    zeros_n = jnp.zeros((NPAD, H), jnp.float32)
    sqh = [sq[:EH], sq[EH:]]
    for l in range(L):
        AS0, BR0 = _gather_h(As, Ar, send3h[0], rec3h[0])
        AS1, BR1 = _gather_h(As, Ar, send3h[1], rec3h[1])
        m0 = _msg_call(AS0, BR0, sqh[0], wd[l], msg_W2[l], msg_b2[l][None])
        pa = _scatter_h(m0, rec3h[0], zeros_n)
        m1 = _msg_call(AS1, BR1, sqh[1], wd[l], msg_W2[l], msg_b2[l][None])
        pb = _scatter_h(m1, rec3h[1], zeros_n)
        if l < L - 1:
            h, As, Ar = _upd_call(
                h, pa[0], pa[1], pb[0], pb[1], U1h[l], U1a[l], ub1[l],
                upd_W2[l], ub2[l], W1s[l + 1], W1r[l + 1], b1m[l + 1])
        else:
            h = _upd_last_call(
                h, pa[0], pa[1], pb[0], pb[1], U1h[l], U1a[l], ub1[l],
                upd_W2[l], ub2[l])

    out = _final_call(
        h, batch[:, None], pre_W1, pre_b1[None], pre_W2, pre_b2[None],
        ro_W1, ro_b1[None], ro_W2, ro_b2[None])
    return jnp.squeeze(out)


# speedup vs baseline: 1.2576x; 1.0936x over previous
import functools

import jax
import jax.numpy as jnp
from jax import lax
from jax.experimental import pallas as pl
from jax.experimental.pallas import tpu as pltpu
from jax.experimental.pallas import tpu_sc as plsc

N = 10000
E = 320000
H = 128
G = 16
L = 4
BLK_N = 2000
BLK_E = 4000

NC = 2
NS = 16
NW = NC * NS
EPW = E // NW
CH = 80
NCH = EPW // CH
NPAD = 10240
RPT = NPAD // NS


def _silu(x):
    return x / (1.0 + jnp.exp(-x))


def _mm(a, b):
    return jax.lax.dot_general(
        a, b, (((1,), (0,)), ((), ())), preferred_element_type=jnp.float32
    )


def _embed_body(xpe, eW1, eb1, eW2, eb2, W1s, W1r, b1m, h_out, as_out, ar_out):
    t = _silu(_mm(xpe[...], eW1[...]) + eb1[...])
    h = _mm(t, eW2[...]) + eb2[...]
    h_out[...] = h
    as_out[...] = _mm(h, W1s[...]) + b1m[...]
    ar_out[...] = _mm(h, W1r[...])


def _upd_body(h, p0, p1, U1h, U1a, ub1, U2, ub2, W1s, W1r, b1m,
              h_out, as_out, ar_out):
    aggr = p0[...] + p1[...]
    t = _silu(_mm(h[...], U1h[...]) + _mm(aggr, U1a[...]) + ub1[...])
    hn = h[...] + _mm(t, U2[...]) + ub2[...]
    h_out[...] = hn
    as_out[...] = _mm(hn, W1s[...]) + b1m[...]
    ar_out[...] = _mm(hn, W1r[...])


def _upd_last_body(h, p0, p1, U1h, U1a, ub1, U2, ub2, h_out):
    aggr = p0[...] + p1[...]
    t = _silu(_mm(h[...], U1h[...]) + _mm(aggr, U1a[...]) + ub1[...])
    h_out[...] = h[...] + _mm(t, U2[...]) + ub2[...]


def _msg_body(asg, brg, sq, wd, W2, b2, m_out):
    s = sq[...]
    dist = jnp.where(s > 0, jnp.sqrt(jnp.where(s > 0, s, 1.0)), 0.0)
    pre = asg[...] + brg[...] + dist * wd[...]
    t = _silu(pre)
    m_out[...] = _silu(_mm(t, W2[...]) + b2[...])


def _final_body(h, bt, pW1, pb1, pW2, pb2, rW1, rb1, rW2, rb2, out, acc):
    i = pl.program_id(0)

    @pl.when(i == 0)
    def _():
        acc[...] = jnp.zeros_like(acc)

    t = _silu(_mm(h[...], pW1[...]) + pb1[...])
    hp = _mm(t, pW2[...]) + pb2[...]
    oh = (bt[...] == jax.lax.broadcasted_iota(jnp.int32, (BLK_N, G), 1)
          ).astype(jnp.float32)
    acc[...] += jax.lax.dot_general(
        oh, hp, (((0,), (0,)), ((), ())), preferred_element_type=jnp.float32
    )

    @pl.when(i == pl.num_programs(0) - 1)
    def _():
        p = acc[...]
        t2 = _silu(_mm(p, rW1[...]) + rb1[...])
        out[...] = _mm(t2, rW2[...]) + rb2[...]


_SC_MESH = plsc.VectorSubcoreMesh(core_axis_name="c", subcore_axis_name="s")


def _make_gather(ne, ch):
  epw = ne // NW
  nch = epw // ch

  def _gather_body(tabS, tabR, send3, rec3, out0, out1, idx_s, idx_r,
                   bSA, bRA, bSB, bRB, gsa, gsb, wsa, wsb):
    wid = lax.axis_index("c") * NS + lax.axis_index("s")
    base = wid * epw
    pltpu.sync_copy(send3.at[wid], idx_s)
    pltpu.sync_copy(rec3.at[wid], idx_r)

    def fire_g(k, bS, bR, sem):
        pltpu.async_copy(tabS.at[idx_s.at[k]], bS, sem)
        pltpu.async_copy(tabR.at[idx_r.at[k]], bR, sem)

    def wait_g(k, bS, bR, sem):
        pltpu.make_async_copy(tabS.at[idx_s.at[k]], bS, sem).wait()
        pltpu.make_async_copy(tabR.at[idx_r.at[k]], bR, sem).wait()

    def fire_w(k, bS, bR, sem):
        pltpu.async_copy(bS, out0.at[pl.ds(base + k * ch, ch)], sem)
        pltpu.async_copy(bR, out1.at[pl.ds(base + k * ch, ch)], sem)

    def wait_w(k, bS, bR, sem):
        pltpu.make_async_copy(
            bS, out0.at[pl.ds(base + k * ch, ch)], sem).wait()
        pltpu.make_async_copy(
            bR, out1.at[pl.ds(base + k * ch, ch)], sem).wait()

    fire_g(0, bSA, bRA, gsa)
    fire_g(1, bSB, bRB, gsb)

    def step(kk, carry):
        k0 = 2 * kk
        k1 = k0 + 1
        wait_g(k0, bSA, bRA, gsa)
        fire_w(k0, bSA, bRA, wsa)
        wait_g(k1, bSB, bRB, gsb)
        fire_w(k1, bSB, bRB, wsb)
        wait_w(k0, bSA, bRA, wsa)
        fire_g(k0 + 2, bSA, bRA, gsa)
        wait_w(k1, bSB, bRB, wsb)

        @pl.when(kk < (nch - 1) // 2 - 1)
        def _():
            fire_g(k1 + 2, bSB, bRB, gsb)

        return carry

    lax.fori_loop(0, (nch - 1) // 2, step, 0)
    k_last = nch - 1
    wait_g(k_last, bSA, bRA, gsa)
    fire_w(k_last, bSA, bRA, wsa)
    wait_w(k_last, bSA, bRA, wsa)

  return pl.kernel(
      _gather_body,
      mesh=_SC_MESH,
      out_type=[jax.ShapeDtypeStruct((ne, H), jnp.float32)] * 2,
      scratch_types=[
          pltpu.VMEM((nch, ch), jnp.int32),
          pltpu.VMEM((nch, ch), jnp.int32),
          pltpu.VMEM((ch, H), jnp.float32),
          pltpu.VMEM((ch, H), jnp.float32),
          pltpu.VMEM((ch, H), jnp.float32),
          pltpu.VMEM((ch, H), jnp.float32),
          pltpu.SemaphoreType.DMA,
          pltpu.SemaphoreType.DMA,
          pltpu.SemaphoreType.DMA,
          pltpu.SemaphoreType.DMA,
      ],
  )


_gather_full = _make_gather(E, CH)

_GPC = CH // 16


def _sq_body(px_h, py_h, pz_h, send3, rec3, sq_hbm, px, py, pz,
             idx_s, idx_r, buf):
    wid = lax.axis_index("c") * NS + lax.axis_index("s")
    base = wid * EPW
    pltpu.sync_copy(px_h, px)
    pltpu.sync_copy(py_h, py)
    pltpu.sync_copy(pz_h, pz)
    pltpu.sync_copy(send3.at[wid], idx_s)
    pltpu.sync_copy(rec3.at[wid], idx_r)

    def chunk(k, carry):
        for j in range(_GPC):
            ivs = idx_s[k, pl.ds(j * 16, 16)]
            ivr = idx_r[k, pl.ds(j * 16, 16)]
            dx = plsc.load_gather(px, [ivs]) - plsc.load_gather(px, [ivr])
            dy = plsc.load_gather(py, [ivs]) - plsc.load_gather(py, [ivr])
            dz = plsc.load_gather(pz, [ivs]) - plsc.load_gather(pz, [ivr])
            buf[pl.ds(j * 16, 16)] = dx * dx + dy * dy + dz * dz
        pltpu.sync_copy(buf, sq_hbm.at[pl.ds(base + k * CH, CH)])
        return carry

    lax.fori_loop(0, NCH, chunk, 0)


_sq_call = pl.kernel(
    _sq_body,
    mesh=_SC_MESH,
    compiler_params=pltpu.CompilerParams(needs_layout_passes=False),
    out_type=jax.ShapeDtypeStruct((E,), jnp.float32),
    scratch_types=[
        pltpu.VMEM((N,), jnp.float32),
        pltpu.VMEM((N,), jnp.float32),
        pltpu.VMEM((N,), jnp.float32),
        pltpu.VMEM((NCH, CH), jnp.int32),
        pltpu.VMEM((NCH, CH), jnp.int32),
        pltpu.VMEM((CH,), jnp.float32),
    ],
)


def _make_scatter(ne, ch):
  epw = ne // NW
  nch = epw // ch

  def _scatter_body(m_hbm, rec3, zeros_hbm, out_hbm, idx_r, bufA, bufB,
                    aggr_sh, sA, sB):
    cid = lax.axis_index("c")
    sid = lax.axis_index("s")
    wid = cid * NS + sid
    base = wid * epw
    pltpu.sync_copy(zeros_hbm.at[pl.ds(sid * RPT, RPT)],
                    aggr_sh.at[pl.ds(sid * RPT, RPT)])
    pltpu.sync_copy(rec3.at[wid], idx_r)
    plsc.subcore_barrier()

    def fire(k, buf, sem):
        pltpu.async_copy(m_hbm.at[pl.ds(base + k * ch, ch)], buf, sem)

    def wait(k, buf, sem):
        pltpu.make_async_copy(
            m_hbm.at[pl.ds(base + k * ch, ch)], buf, sem).wait()

    fire(0, bufA, sA)

    def step(kk, carry):
        k0 = 2 * kk
        k1 = k0 + 1
        wait(k0, bufA, sA)
        fire(k1, bufB, sB)
        pltpu.sync_copy(bufA, aggr_sh.at[idx_r.at[k0]], add=True)
        wait(k1, bufB, sB)
        fire(k0 + 2, bufA, sA)
        pltpu.sync_copy(bufB, aggr_sh.at[idx_r.at[k1]], add=True)
        return carry

    lax.fori_loop(0, (nch - 1) // 2, step, 0)
    k_last = nch - 1
    wait(k_last, bufA, sA)
    pltpu.sync_copy(bufA, aggr_sh.at[idx_r.at[k_last]], add=True)
    plsc.subcore_barrier()
    pltpu.sync_copy(aggr_sh.at[pl.ds(sid * RPT, RPT)],
                    out_hbm.at[cid, pl.ds(sid * RPT, RPT)])

  return pl.kernel(
      _scatter_body,
      mesh=_SC_MESH,
      out_type=jax.ShapeDtypeStruct((NC, NPAD, H), jnp.float32),
      scratch_types=[
          pltpu.VMEM((nch, ch), jnp.int32),
          pltpu.VMEM((ch, H), jnp.float32),
          pltpu.VMEM((ch, H), jnp.float32),
          pltpu.VMEM_SHARED((NPAD, H), jnp.float32),
          pltpu.SemaphoreType.DMA,
          pltpu.SemaphoreType.DMA,
      ],
  )


_scatter_full = _make_scatter(E, CH)


def _full(shape):
    return pl.BlockSpec(shape, lambda i: (0,) * len(shape))


def _nrows(width):
    return pl.BlockSpec((BLK_N, width), lambda i: (i, 0))


def _erows(width):
    return pl.BlockSpec((BLK_E, width), lambda i: (i, 0))


_W = _full((H, H))
_B = _full((1, H))

_embed_call = pl.pallas_call(
    _embed_body,
    grid=(N // BLK_N,),
    in_specs=[_nrows(H), _W, _B, _W, _B, _W, _W, _B],
    out_specs=[_nrows(H), _nrows(H), _nrows(H)],
    out_shape=[jax.ShapeDtypeStruct((N, H), jnp.float32)] * 3,
)

_upd_call = pl.pallas_call(
    _upd_body,
    grid=(N // BLK_N,),
    in_specs=[_nrows(H)] * 3 + [_W, _W, _B, _W, _B, _W, _W, _B],
    out_specs=[_nrows(H), _nrows(H), _nrows(H)],
    out_shape=[jax.ShapeDtypeStruct((N, H), jnp.float32)] * 3,
)

_upd_last_call = pl.pallas_call(
    _upd_last_body,
    grid=(N // BLK_N,),
    in_specs=[_nrows(H)] * 3 + [_W, _W, _B, _W, _B],
    out_specs=_nrows(H),
    out_shape=jax.ShapeDtypeStruct((N, H), jnp.float32),
)

_msg_call = pl.pallas_call(
    _msg_body,
    grid=(E // BLK_E,),
    in_specs=[_erows(H), _erows(H), _erows(1), _B, _W, _B],
    out_specs=_erows(H),
    out_shape=jax.ShapeDtypeStruct((E, H), jnp.float32),
)

_final_call = pl.pallas_call(
    _final_body,
    grid=(N // BLK_N,),
    in_specs=[_nrows(H), _nrows(1), _W, _B, _W, _B, _W, _B, _full((H, 1)),
              _full((1, 1))],
    out_specs=_full((G, 1)),
    out_shape=jax.ShapeDtypeStruct((G, 1), jnp.float32),
    scratch_shapes=[pltpu.VMEM((G, H), jnp.float32)],
)


def kernel(x, pos, pe, edge_index, batch,
           embed_W1, embed_b1, embed_W2, embed_b2,
           msg_W1, msg_b1, msg_W2, msg_b2,
           upd_W1, upd_b1, upd_W2, upd_b2,
           pre_W1, pre_b1, pre_W2, pre_b2,
           ro_W1, ro_b1, ro_W2, ro_b2):
    xpe = jnp.concatenate([x, pe], axis=1)
    send, rec = edge_index[0], edge_index[1]
    send3 = send.reshape(NW, NCH, CH)
    rec3 = rec.reshape(NW, NCH, CH)

    sq = _sq_call(pos[:, 0], pos[:, 1], pos[:, 2], send3, rec3)[:, None]

    W1s = [msg_W1[l, :H] for l in range(L)]
    W1r = [msg_W1[l, H:2 * H] for l in range(L)]
    wd = [msg_W1[l, 2 * H][None] for l in range(L)]
    b1m = [msg_b1[l][None] for l in range(L)]
    U1h = [upd_W1[l, :H] for l in range(L)]
    U1a = [upd_W1[l, H:] for l in range(L)]
    ub1 = [upd_b1[l][None] for l in range(L)]
    ub2 = [upd_b2[l][None] for l in range(L)]

    h, As, Ar = _embed_call(
        xpe, embed_W1, embed_b1[None], embed_W2, embed_b2[None],
        W1s[0], W1r[0], b1m[0])

    zeros_n = jnp.zeros((NPAD, H), jnp.float32)
    for l in range(L):
        AS, BR = _gather_full(As, Ar, send3, rec3)
        m = _msg_call(AS, BR, sq, wd[l], msg_W2[l], msg_b2[l][None])
        parts = _scatter_full(m, rec3, zeros_n)
        if l < L - 1:
            h, As, Ar = _upd_call(
                h, parts[0], parts[1], U1h[l], U1a[l], ub1[l],
                upd_W2[l], ub2[l], W1s[l + 1], W1r[l + 1], b1m[l + 1])
        else:
            h = _upd_last_call(
                h, parts[0], parts[1], U1h[l], U1a[l], ub1[l],
                upd_W2[l], ub2[l])

    out = _final_call(
        h, batch[:, None], pre_W1, pre_b1[None], pre_W2, pre_b2[None],
        ro_W1, ro_b1[None], ro_W2, ro_b2[None])
    return jnp.squeeze(out)


# speedup vs baseline: 1.2708x; 1.0105x over previous
import functools

import jax
import jax.numpy as jnp
from jax import lax
from jax.experimental import pallas as pl
from jax.experimental.pallas import tpu as pltpu
from jax.experimental.pallas import tpu_sc as plsc

N = 10000
E = 320000
H = 128
G = 16
L = 4
BLK_N = 2000
BLK_E = 8000

NC = 2
NS = 16
NW = NC * NS
EPW = E // NW
CH = 80
NCH = EPW // CH
NPAD = 10240
RPT = NPAD // NS


def _silu(x):
    return x / (1.0 + jnp.exp(-x))


def _mm(a, b):
    return jax.lax.dot_general(
        a, b, (((1,), (0,)), ((), ())), preferred_element_type=jnp.float32
    )


def _embed_body(xpe, eW1, eb1, eW2, eb2, W1s, W1r, b1m, h_out, as_out, ar_out):
    t = _silu(_mm(xpe[...], eW1[...]) + eb1[...])
    h = _mm(t, eW2[...]) + eb2[...]
    h_out[...] = h
    as_out[...] = _mm(h, W1s[...]) + b1m[...]
    ar_out[...] = _mm(h, W1r[...])


def _upd_body(h, p0, p1, U1h, U1a, ub1, U2, ub2, W1s, W1r, b1m,
              h_out, as_out, ar_out):
    aggr = p0[...] + p1[...]
    t = _silu(_mm(h[...], U1h[...]) + _mm(aggr, U1a[...]) + ub1[...])
    hn = h[...] + _mm(t, U2[...]) + ub2[...]
    h_out[...] = hn
    as_out[...] = _mm(hn, W1s[...]) + b1m[...]
    ar_out[...] = _mm(hn, W1r[...])


def _upd_last_body(h, p0, p1, U1h, U1a, ub1, U2, ub2, h_out):
    aggr = p0[...] + p1[...]
    t = _silu(_mm(h[...], U1h[...]) + _mm(aggr, U1a[...]) + ub1[...])
    h_out[...] = h[...] + _mm(t, U2[...]) + ub2[...]


def _msg_body(asg, brg, sq, wd, W2, b2, m_out):
    s = sq[...]
    dist = jnp.where(s > 0, jnp.sqrt(jnp.where(s > 0, s, 1.0)), 0.0)
    pre = asg[...] + brg[...] + dist * wd[...]
    t = _silu(pre)
    m_out[...] = _silu(_mm(t, W2[...]) + b2[...])


def _final_body(h, bt, pW1, pb1, pW2, pb2, rW1, rb1, rW2, rb2, out, acc):
    i = pl.program_id(0)

    @pl.when(i == 0)
    def _():
        acc[...] = jnp.zeros_like(acc)

    t = _silu(_mm(h[...], pW1[...]) + pb1[...])
    hp = _mm(t, pW2[...]) + pb2[...]
    oh = (bt[...] == jax.lax.broadcasted_iota(jnp.int32, (BLK_N, G), 1)
          ).astype(jnp.float32)
    acc[...] += jax.lax.dot_general(
        oh, hp, (((0,), (0,)), ((), ())), preferred_element_type=jnp.float32
    )

    @pl.when(i == pl.num_programs(0) - 1)
    def _():
        p = acc[...]
        t2 = _silu(_mm(p, rW1[...]) + rb1[...])
        out[...] = _mm(t2, rW2[...]) + rb2[...]


_SC_MESH = plsc.VectorSubcoreMesh(core_axis_name="c", subcore_axis_name="s")


def _make_gather(ne, ch):
  epw = ne // NW
  nch = epw // ch

  def _gather_body(tabS, tabR, send3, rec3, out0, out1, idx_s, idx_r,
                   bSA, bRA, bSB, bRB, gsa, gsb, wsa, wsb):
    wid = lax.axis_index("c") * NS + lax.axis_index("s")
    base = wid * epw
    pltpu.sync_copy(send3.at[wid], idx_s)
    pltpu.sync_copy(rec3.at[wid], idx_r)

    def fire_g(k, bS, bR, sem):
        pltpu.async_copy(tabS.at[idx_s.at[k]], bS, sem)
        pltpu.async_copy(tabR.at[idx_r.at[k]], bR, sem)

    def wait_g(k, bS, bR, sem):
        pltpu.make_async_copy(tabS.at[idx_s.at[k]], bS, sem).wait()
        pltpu.make_async_copy(tabR.at[idx_r.at[k]], bR, sem).wait()

    def fire_w(k, bS, bR, sem):
        pltpu.async_copy(bS, out0.at[pl.ds(base + k * ch, ch)], sem)
        pltpu.async_copy(bR, out1.at[pl.ds(base + k * ch, ch)], sem)

    def wait_w(k, bS, bR, sem):
        pltpu.make_async_copy(
            bS, out0.at[pl.ds(base + k * ch, ch)], sem).wait()
        pltpu.make_async_copy(
            bR, out1.at[pl.ds(base + k * ch, ch)], sem).wait()

    fire_g(0, bSA, bRA, gsa)
    fire_g(1, bSB, bRB, gsb)

    def step(kk, carry):
        k0 = 2 * kk
        k1 = k0 + 1
        wait_g(k0, bSA, bRA, gsa)
        fire_w(k0, bSA, bRA, wsa)
        wait_g(k1, bSB, bRB, gsb)
        fire_w(k1, bSB, bRB, wsb)
        wait_w(k0, bSA, bRA, wsa)
        fire_g(k0 + 2, bSA, bRA, gsa)
        wait_w(k1, bSB, bRB, wsb)

        @pl.when(kk < (nch - 1) // 2 - 1)
        def _():
            fire_g(k1 + 2, bSB, bRB, gsb)

        return carry

    lax.fori_loop(0, (nch - 1) // 2, step, 0)
    k_last = nch - 1
    wait_g(k_last, bSA, bRA, gsa)
    fire_w(k_last, bSA, bRA, wsa)
    wait_w(k_last, bSA, bRA, wsa)

  return pl.kernel(
      _gather_body,
      mesh=_SC_MESH,
      out_type=[jax.ShapeDtypeStruct((ne, H), jnp.float32)] * 2,
      scratch_types=[
          pltpu.VMEM((nch, ch), jnp.int32),
          pltpu.VMEM((nch, ch), jnp.int32),
          pltpu.VMEM((ch, H), jnp.float32),
          pltpu.VMEM((ch, H), jnp.float32),
          pltpu.VMEM((ch, H), jnp.float32),
          pltpu.VMEM((ch, H), jnp.float32),
          pltpu.SemaphoreType.DMA,
          pltpu.SemaphoreType.DMA,
          pltpu.SemaphoreType.DMA,
          pltpu.SemaphoreType.DMA,
      ],
  )


_gather_full = _make_gather(E, CH)

_GPC = CH // 16


def _sq_body(px_h, py_h, pz_h, send3, rec3, sq_hbm, px, py, pz,
             idx_s, idx_r, buf):
    wid = lax.axis_index("c") * NS + lax.axis_index("s")
    base = wid * EPW
    pltpu.sync_copy(px_h, px)
    pltpu.sync_copy(py_h, py)
    pltpu.sync_copy(pz_h, pz)
    pltpu.sync_copy(send3.at[wid], idx_s)
    pltpu.sync_copy(rec3.at[wid], idx_r)

    def chunk(k, carry):
        for j in range(_GPC):
            ivs = idx_s[k, pl.ds(j * 16, 16)]
            ivr = idx_r[k, pl.ds(j * 16, 16)]
            dx = plsc.load_gather(px, [ivs]) - plsc.load_gather(px, [ivr])
            dy = plsc.load_gather(py, [ivs]) - plsc.load_gather(py, [ivr])
            dz = plsc.load_gather(pz, [ivs]) - plsc.load_gather(pz, [ivr])
            buf[pl.ds(j * 16, 16)] = dx * dx + dy * dy + dz * dz
        pltpu.sync_copy(buf, sq_hbm.at[pl.ds(base + k * CH, CH)])
        return carry

    lax.fori_loop(0, NCH, chunk, 0)


_sq_call = pl.kernel(
    _sq_body,
    mesh=_SC_MESH,
    compiler_params=pltpu.CompilerParams(needs_layout_passes=False),
    out_type=jax.ShapeDtypeStruct((E,), jnp.float32),
    scratch_types=[
        pltpu.VMEM((N,), jnp.float32),
        pltpu.VMEM((N,), jnp.float32),
        pltpu.VMEM((N,), jnp.float32),
        pltpu.VMEM((NCH, CH), jnp.int32),
        pltpu.VMEM((NCH, CH), jnp.int32),
        pltpu.VMEM((CH,), jnp.float32),
    ],
)


def _make_scatter(ne, ch):
  epw = ne // NW
  nch = epw // ch

  def _scatter_body(m_hbm, rec3, zeros_hbm, out_hbm, idx_r, bufA, bufB,
                    aggr_sh, sA, sB):
    cid = lax.axis_index("c")
    sid = lax.axis_index("s")
    wid = cid * NS + sid
    base = wid * epw
    pltpu.sync_copy(zeros_hbm.at[pl.ds(sid * RPT, RPT)],
                    aggr_sh.at[pl.ds(sid * RPT, RPT)])
    pltpu.sync_copy(rec3.at[wid], idx_r)
    plsc.subcore_barrier()

    def fire(k, buf, sem):
        pltpu.async_copy(m_hbm.at[pl.ds(base + k * ch, ch)], buf, sem)

    def wait(k, buf, sem):
        pltpu.make_async_copy(
            m_hbm.at[pl.ds(base + k * ch, ch)], buf, sem).wait()

    fire(0, bufA, sA)

    def step(kk, carry):
        k0 = 2 * kk
        k1 = k0 + 1
        wait(k0, bufA, sA)
        fire(k1, bufB, sB)
        pltpu.sync_copy(bufA, aggr_sh.at[idx_r.at[k0]], add=True)
        wait(k1, bufB, sB)
        fire(k0 + 2, bufA, sA)
        pltpu.sync_copy(bufB, aggr_sh.at[idx_r.at[k1]], add=True)
        return carry

    lax.fori_loop(0, (nch - 1) // 2, step, 0)
    k_last = nch - 1
    wait(k_last, bufA, sA)
    pltpu.sync_copy(bufA, aggr_sh.at[idx_r.at[k_last]], add=True)
    plsc.subcore_barrier()
    pltpu.sync_copy(aggr_sh.at[pl.ds(sid * RPT, RPT)],
                    out_hbm.at[cid, pl.ds(sid * RPT, RPT)])

  return pl.kernel(
      _scatter_body,
      mesh=_SC_MESH,
      out_type=jax.ShapeDtypeStruct((NC, NPAD, H), jnp.float32),
      scratch_types=[
          pltpu.VMEM((nch, ch), jnp.int32),
          pltpu.VMEM((ch, H), jnp.float32),
          pltpu.VMEM((ch, H), jnp.float32),
          pltpu.VMEM_SHARED((NPAD, H), jnp.float32),
          pltpu.SemaphoreType.DMA,
          pltpu.SemaphoreType.DMA,
      ],
  )


_scatter_full = _make_scatter(E, CH)


def _full(shape):
    return pl.BlockSpec(shape, lambda i: (0,) * len(shape))


def _nrows(width):
    return pl.BlockSpec((BLK_N, width), lambda i: (i, 0))


def _erows(width):
    return pl.BlockSpec((BLK_E, width), lambda i: (i, 0))


_W = _full((H, H))
_B = _full((1, H))

_embed_call = pl.pallas_call(
    _embed_body,
    grid=(N // BLK_N,),
    in_specs=[_nrows(H), _W, _B, _W, _B, _W, _W, _B],
    out_specs=[_nrows(H), _nrows(H), _nrows(H)],
    out_shape=[jax.ShapeDtypeStruct((N, H), jnp.float32)] * 3,
)

_upd_call = pl.pallas_call(
    _upd_body,
    grid=(N // BLK_N,),
    in_specs=[_nrows(H)] * 3 + [_W, _W, _B, _W, _B, _W, _W, _B],
    out_specs=[_nrows(H), _nrows(H), _nrows(H)],
    out_shape=[jax.ShapeDtypeStruct((N, H), jnp.float32)] * 3,
)

_upd_last_call = pl.pallas_call(
    _upd_last_body,
    grid=(N // BLK_N,),
    in_specs=[_nrows(H)] * 3 + [_W, _W, _B, _W, _B],
    out_specs=_nrows(H),
    out_shape=jax.ShapeDtypeStruct((N, H), jnp.float32),
)

_msg_call = pl.pallas_call(
    _msg_body,
    grid=(E // BLK_E,),
    in_specs=[_erows(H), _erows(H), _erows(1), _B, _W, _B],
    out_specs=_erows(H),
    out_shape=jax.ShapeDtypeStruct((E, H), jnp.float32),
)

_final_call = pl.pallas_call(
    _final_body,
    grid=(N // BLK_N,),
    in_specs=[_nrows(H), _nrows(1), _W, _B, _W, _B, _W, _B, _full((H, 1)),
              _full((1, 1))],
    out_specs=_full((G, 1)),
    out_shape=jax.ShapeDtypeStruct((G, 1), jnp.float32),
    scratch_shapes=[pltpu.VMEM((G, H), jnp.float32)],
)


def kernel(x, pos, pe, edge_index, batch,
           embed_W1, embed_b1, embed_W2, embed_b2,
           msg_W1, msg_b1, msg_W2, msg_b2,
           upd_W1, upd_b1, upd_W2, upd_b2,
           pre_W1, pre_b1, pre_W2, pre_b2,
           ro_W1, ro_b1, ro_W2, ro_b2):
    xpe = jnp.concatenate([x, pe], axis=1)
    send, rec = edge_index[0], edge_index[1]
    send3 = send.reshape(NW, NCH, CH)
    rec3 = rec.reshape(NW, NCH, CH)

    sq = _sq_call(pos[:, 0], pos[:, 1], pos[:, 2], send3, rec3)[:, None]

    W1s = [msg_W1[l, :H] for l in range(L)]
    W1r = [msg_W1[l, H:2 * H] for l in range(L)]
    wd = [msg_W1[l, 2 * H][None] for l in range(L)]
    b1m = [msg_b1[l][None] for l in range(L)]
    U1h = [upd_W1[l, :H] for l in range(L)]
    U1a = [upd_W1[l, H:] for l in range(L)]
    ub1 = [upd_b1[l][None] for l in range(L)]
    ub2 = [upd_b2[l][None] for l in range(L)]

    h, As, Ar = _embed_call(
        xpe, embed_W1, embed_b1[None], embed_W2, embed_b2[None],
        W1s[0], W1r[0], b1m[0])

    zeros_n = jnp.zeros((NPAD, H), jnp.float32)
    for l in range(L):
        AS, BR = _gather_full(As, Ar, send3, rec3)
        m = _msg_call(AS, BR, sq, wd[l], msg_W2[l], msg_b2[l][None])
        parts = _scatter_full(m, rec3, zeros_n)
        if l < L - 1:
            h, As, Ar = _upd_call(
                h, parts[0], parts[1], U1h[l], U1a[l], ub1[l],
                upd_W2[l], ub2[l], W1s[l + 1], W1r[l + 1], b1m[l + 1])
        else:
            h = _upd_last_call(
                h, parts[0], parts[1], U1h[l], U1a[l], ub1[l],
                upd_W2[l], ub2[l])

    out = _final_call(
        h, batch[:, None], pre_W1, pre_b1[None], pre_W2, pre_b2[None],
        ro_W1, ro_b1[None], ro_W2, ro_b2[None])
    return jnp.squeeze(out)
